# baseline (speedup 1.0000x reference)
import jax
import jax.numpy as jnp
from jax import lax
from jax.experimental import pallas as pl
from jax.experimental.pallas import tpu as pltpu
from jax.experimental.pallas import tpu_sc as plsc

N = 10000
E = 320000
D = 128
DH = 64

NC = 2
NS = 16
NW = NC * NS

CHUNK = 128
NCH = 79
NCHG = NCH + 2
EPT = NCH * CHUNK
NPAD = 10112
ZROWS = NPAD // NS


def _sc_segment_sum_body(y_hbm, src_hbm, dst_hbm, zeros_hbm, part_hbm,
                         sidx, didx, rows, ytab, accum, semg, sems):
    c = lax.axis_index("c")
    s = lax.axis_index("s")
    wid = s * NC + c

    z0 = s * ZROWS
    pltpu.async_copy(src_hbm.at[wid], sidx, sems)
    pltpu.async_copy(dst_hbm.at[wid], didx, sems)
    pltpu.async_copy(y_hbm.at[pl.ds(z0, ZROWS)], ytab.at[pl.ds(z0, ZROWS)], sems)
    pltpu.async_copy(zeros_hbm.at[pl.ds(z0, ZROWS)], accum.at[pl.ds(z0, ZROWS)],
                     sems)
    pltpu.make_async_copy(src_hbm.at[wid], sidx, sems).wait()
    pltpu.make_async_copy(dst_hbm.at[wid], didx, sems).wait()
    pltpu.make_async_copy(y_hbm.at[pl.ds(z0, ZROWS)], ytab.at[pl.ds(z0, ZROWS)],
                          sems).wait()
    pltpu.make_async_copy(zeros_hbm.at[pl.ds(z0, ZROWS)],
                          accum.at[pl.ds(z0, ZROWS)], sems).wait()
    plsc.subcore_barrier()

    pltpu.async_copy(ytab.at[sidx.at[0]], rows.at[0], semg)
    pltpu.async_copy(ytab.at[sidx.at[1]], rows.at[1], semg)

    def chunk_body(j, carry):
        pltpu.async_copy(ytab.at[sidx.at[j + 2]], rows.at[lax.rem(j + 2, 3)],
                         semg)
        pltpu.make_async_copy(ytab.at[sidx.at[0]], rows.at[0], semg).wait()
        pltpu.sync_copy(rows.at[lax.rem(j, 3)], accum.at[didx.at[j]], add=True)
        return carry

    lax.fori_loop(0, NCH, chunk_body, 0)
    pltpu.make_async_copy(ytab.at[sidx.at[0]], rows.at[0], semg).wait()
    pltpu.make_async_copy(ytab.at[sidx.at[0]], rows.at[0], semg).wait()
    plsc.subcore_barrier()

    pltpu.sync_copy(accum.at[pl.ds(z0, ZROWS)], part_hbm.at[c, pl.ds(z0, ZROWS)])


_sc_segment_sum = pl.kernel(
    _sc_segment_sum_body,
    out_type=jax.ShapeDtypeStruct((NC, NPAD, DH), jnp.float32),
    mesh=plsc.VectorSubcoreMesh(
        core_axis_name="c", subcore_axis_name="s", num_cores=NC, num_subcores=NS
    ),
    scratch_types=[
        pltpu.VMEM((NCHG, CHUNK), jnp.int32),
        pltpu.VMEM((NCHG, CHUNK), jnp.int32),
        pltpu.VMEM((3, CHUNK, DH), jnp.float32),
        pltpu.VMEM_SHARED((NPAD, DH), jnp.float32),
        pltpu.VMEM_SHARED((NPAD, DH), jnp.float32),
        pltpu.SemaphoreType.DMA,
        pltpu.SemaphoreType.DMA,
    ],
    compiler_params=pltpu.CompilerParams(
        use_tc_tiling_on_sc=False, disable_bounds_checks=True
    ),
)


def _tc_mid_body(part_ref, x_ref, w_ref, b_ref, o_ref):
    agg = part_ref[0] + part_ref[1]
    h = jnp.tanh(jnp.dot(agg, w_ref[...], preferred_element_type=jnp.float32)
                 + b_ref[...])
    o_ref[:N] = x_ref[:, :DH] + h[:N]
    o_ref[N:] = h[N:]


def _tc_mid(part, x, w, b):
    return pl.pallas_call(
        _tc_mid_body,
        out_shape=jax.ShapeDtypeStruct((NPAD, DH), jnp.float32),
    )(part, x, w, b.reshape(1, DH))


def _tc_final_body(part_ref, x_ref, y0_ref, w_ref, b_ref, o_ref):
    agg = part_ref[0, :N] + part_ref[1, :N]
    h = jnp.tanh(jnp.dot(agg, w_ref[...], preferred_element_type=jnp.float32)
                 + b_ref[...])
    o_ref[:, :DH] = y0_ref[:N]
    o_ref[:, DH:] = x_ref[:, DH:] + h


def _tc_final(part, x, y0, w, b):
    return pl.pallas_call(
        _tc_final_body,
        out_shape=jax.ShapeDtypeStruct((N, D), jnp.float32),
    )(part, x, y0, w, b.reshape(1, DH))


@jax.jit
def kernel(x, edge_index, W0, b0, W1, b1):
    x1 = jnp.pad(x[:, DH:], ((0, NPAD - N), (0, 0)))
    pad = NW * EPT - E
    src = jnp.concatenate([edge_index[0], jnp.zeros((pad,), jnp.int32)])
    dst = jnp.concatenate(
        [edge_index[1], N + (jnp.arange(pad, dtype=jnp.int32) % (NPAD - N))])
    dummy = jnp.zeros((NW, NCHG - NCH, CHUNK), jnp.int32)
    src = jnp.concatenate([src.reshape(NW, NCH, CHUNK), dummy], axis=1)
    dst = jnp.concatenate([dst.reshape(NW, NCH, CHUNK), N + dummy], axis=1)
    zeros = jnp.zeros((NPAD, DH), jnp.float32)

    p0 = _sc_segment_sum(x1, src, dst, zeros)
    y0 = _tc_mid(p0, x, W0, b0)
    p1 = _sc_segment_sum(y0, src, dst, zeros)
    return _tc_final(p1, x, y0, W1, b1)


# speedup vs baseline: 1.0020x; 1.0020x over previous
import jax
import jax.numpy as jnp
from jax import lax
from jax.experimental import pallas as pl
from jax.experimental.pallas import tpu as pltpu
from jax.experimental.pallas import tpu_sc as plsc

N = 10000
E = 320000
D = 128
DH = 64

NC = 2
NS = 16
NW = NC * NS

CHUNK = 128
NCH = 79
NCHG = NCH + 2
EPT = NCH * CHUNK
NPAD = 10112
ZROWS = NPAD // NS


def _sc_segment_sum_body(y_hbm, src_hbm, dst_hbm, zeros_hbm, part_hbm,
                         sidx, didx, rows, ytab, accum, semg, sems):
    c = lax.axis_index("c")
    s = lax.axis_index("s")
    wid = s * NC + c

    z0 = s * ZROWS
    pltpu.async_copy(src_hbm.at[wid], sidx, sems)
    pltpu.async_copy(dst_hbm.at[wid], didx, sems)
    pltpu.async_copy(y_hbm.at[pl.ds(z0, ZROWS)], ytab.at[pl.ds(z0, ZROWS)], sems)
    pltpu.async_copy(zeros_hbm.at[pl.ds(z0, ZROWS)], accum.at[pl.ds(z0, ZROWS)],
                     sems)
    pltpu.make_async_copy(src_hbm.at[wid], sidx, sems).wait()
    pltpu.make_async_copy(dst_hbm.at[wid], didx, sems).wait()
    pltpu.make_async_copy(y_hbm.at[pl.ds(z0, ZROWS)], ytab.at[pl.ds(z0, ZROWS)],
                          sems).wait()
    pltpu.make_async_copy(zeros_hbm.at[pl.ds(z0, ZROWS)],
                          accum.at[pl.ds(z0, ZROWS)], sems).wait()
    plsc.subcore_barrier()

    pltpu.async_copy(ytab.at[sidx.at[0]], rows.at[0], semg)
    pltpu.async_copy(ytab.at[sidx.at[1]], rows.at[1], semg)

    def chunk_body(j, carry):
        @pl.when(j >= 1)
        def _():
            pltpu.make_async_copy(rows.at[0], accum.at[didx.at[0]], sems).wait()

        pltpu.async_copy(ytab.at[sidx.at[j + 2]], rows.at[lax.rem(j + 2, 3)],
                         semg)
        pltpu.make_async_copy(ytab.at[sidx.at[0]], rows.at[0], semg).wait()
        pltpu.async_copy(rows.at[lax.rem(j, 3)], accum.at[didx.at[j]], sems,
                         add=True)
        return carry

    lax.fori_loop(0, NCH, chunk_body, 0)
    pltpu.make_async_copy(rows.at[0], accum.at[didx.at[0]], sems).wait()
    pltpu.make_async_copy(ytab.at[sidx.at[0]], rows.at[0], semg).wait()
    pltpu.make_async_copy(ytab.at[sidx.at[0]], rows.at[0], semg).wait()
    plsc.subcore_barrier()

    pltpu.sync_copy(accum.at[pl.ds(z0, ZROWS)], part_hbm.at[c, pl.ds(z0, ZROWS)])


_sc_segment_sum = pl.kernel(
    _sc_segment_sum_body,
    out_type=jax.ShapeDtypeStruct((NC, NPAD, DH), jnp.float32),
    mesh=plsc.VectorSubcoreMesh(
        core_axis_name="c", subcore_axis_name="s", num_cores=NC, num_subcores=NS
    ),
    scratch_types=[
        pltpu.VMEM((NCHG, CHUNK), jnp.int32),
        pltpu.VMEM((NCHG, CHUNK), jnp.int32),
        pltpu.VMEM((3, CHUNK, DH), jnp.float32),
        pltpu.VMEM_SHARED((NPAD, DH), jnp.float32),
        pltpu.VMEM_SHARED((NPAD, DH), jnp.float32),
        pltpu.SemaphoreType.DMA,
        pltpu.SemaphoreType.DMA,
    ],
    compiler_params=pltpu.CompilerParams(
        use_tc_tiling_on_sc=False, disable_bounds_checks=True
    ),
)


def _tc_mid_body(part_ref, x_ref, w_ref, b_ref, o_ref):
    agg = part_ref[0] + part_ref[1]
    h = jnp.tanh(jnp.dot(agg, w_ref[...], preferred_element_type=jnp.float32)
                 + b_ref[...])
    o_ref[:N] = x_ref[:, :DH] + h[:N]
    o_ref[N:] = h[N:]


def _tc_mid(part, x, w, b):
    return pl.pallas_call(
        _tc_mid_body,
        out_shape=jax.ShapeDtypeStruct((NPAD, DH), jnp.float32),
    )(part, x, w, b.reshape(1, DH))


def _tc_final_body(part_ref, x_ref, y0_ref, w_ref, b_ref, o_ref):
    agg = part_ref[0, :N] + part_ref[1, :N]
    h = jnp.tanh(jnp.dot(agg, w_ref[...], preferred_element_type=jnp.float32)
                 + b_ref[...])
    o_ref[:, :DH] = y0_ref[:N]
    o_ref[:, DH:] = x_ref[:, DH:] + h


def _tc_final(part, x, y0, w, b):
    return pl.pallas_call(
        _tc_final_body,
        out_shape=jax.ShapeDtypeStruct((N, D), jnp.float32),
    )(part, x, y0, w, b.reshape(1, DH))


@jax.jit
def kernel(x, edge_index, W0, b0, W1, b1):
    x1 = jnp.pad(x[:, DH:], ((0, NPAD - N), (0, 0)))
    pad = NW * EPT - E
    src = jnp.concatenate([edge_index[0], jnp.zeros((pad,), jnp.int32)])
    dst = jnp.concatenate(
        [edge_index[1], N + (jnp.arange(pad, dtype=jnp.int32) % (NPAD - N))])
    dummy = jnp.zeros((NW, NCHG - NCH, CHUNK), jnp.int32)
    src = jnp.concatenate([src.reshape(NW, NCH, CHUNK), dummy], axis=1)
    dst = jnp.concatenate([dst.reshape(NW, NCH, CHUNK), N + dummy], axis=1)
    zeros = jnp.zeros((NPAD, DH), jnp.float32)

    p0 = _sc_segment_sum(x1, src, dst, zeros)
    y0 = _tc_mid(p0, x, W0, b0)
    p1 = _sc_segment_sum(y0, src, dst, zeros)
    return _tc_final(p1, x, y0, W1, b1)
